# baseline (speedup 1.0000x reference)
import jax
import jax.numpy as jnp
from jax.experimental import pallas as pl
from jax.experimental.pallas import tpu as pltpu

N_TOK = 32768
FEAT_DIM = 768
TEXT_DIM = 768
NUM_EXPERTS = 8
HID = 128
BLK = 2048


def _router_body(temp_ref, f_ref, t_ref, fmu_ref, frho_ref, tmu_ref, trho_ref,
                 cmu_ref, crho_ref, ef_ref, et_ref, ec_ref,
                 probs_ref, logits_ref, fw_s, tw_s, cw_s):
    @pl.when(pl.program_id(0) == 0)
    def _():
        fw_s[...] = fmu_ref[...] + jnp.log(1.0 + jnp.exp(frho_ref[...])) * ef_ref[...]
        tw_s[...] = tmu_ref[...] + jnp.log(1.0 + jnp.exp(trho_ref[...])) * et_ref[...]
        cw_s[...] = cmu_ref[...] + jnp.log(1.0 + jnp.exp(crho_ref[...])) * ec_ref[...]

    fp = jnp.dot(f_ref[...], fw_s[...], preferred_element_type=jnp.float32)
    tp = jnp.dot(t_ref[...], tw_s[...], preferred_element_type=jnp.float32)
    logits = (jnp.dot(fp, cw_s[:HID, :], preferred_element_type=jnp.float32)
              + jnp.dot(tp, cw_s[HID:, :], preferred_element_type=jnp.float32))
    inv_t = 1.0 / jnp.maximum(temp_ref[0, 0], 0.1)
    logits = logits * inv_t
    logits_ref[...] = logits
    m = jnp.max(logits, axis=1, keepdims=True)
    e = jnp.exp(logits - m)
    probs_ref[...] = e / jnp.sum(e, axis=1, keepdims=True)


def kernel(feature, text_embedding, feature_mu, feature_rho, text_mu, text_rho,
           combined_mu, combined_rho, temperature, epsilon_f, epsilon_t, epsilon_c):
    temp2d = temperature.reshape(1, 1)
    full = lambda shape: pl.BlockSpec(shape, lambda i: (0, 0))
    grid = N_TOK // BLK
    probs, logits = pl.pallas_call(
        _router_body,
        grid=(grid,),
        in_specs=[
            full((1, 1)),
            pl.BlockSpec((BLK, FEAT_DIM), lambda i: (i, 0)),
            pl.BlockSpec((BLK, TEXT_DIM), lambda i: (i, 0)),
            full((FEAT_DIM, HID)),
            full((FEAT_DIM, HID)),
            full((TEXT_DIM, HID)),
            full((TEXT_DIM, HID)),
            full((2 * HID, NUM_EXPERTS)),
            full((2 * HID, NUM_EXPERTS)),
            full((FEAT_DIM, HID)),
            full((TEXT_DIM, HID)),
            full((2 * HID, NUM_EXPERTS)),
        ],
        out_specs=[
            pl.BlockSpec((BLK, NUM_EXPERTS), lambda i: (i, 0)),
            pl.BlockSpec((BLK, NUM_EXPERTS), lambda i: (i, 0)),
        ],
        out_shape=[
            jax.ShapeDtypeStruct((N_TOK, NUM_EXPERTS), jnp.float32),
            jax.ShapeDtypeStruct((N_TOK, NUM_EXPERTS), jnp.float32),
        ],
        scratch_shapes=[
            pltpu.VMEM((FEAT_DIM, HID), jnp.float32),
            pltpu.VMEM((TEXT_DIM, HID), jnp.float32),
            pltpu.VMEM((2 * HID, NUM_EXPERTS), jnp.float32),
        ],
    )(temp2d, feature, text_embedding, feature_mu, feature_rho, text_mu,
      text_rho, combined_mu, combined_rho, epsilon_f, epsilon_t, epsilon_c)
    return (probs, logits)


# speedup vs baseline: 1.0102x; 1.0102x over previous
import jax
import jax.numpy as jnp
from jax.experimental import pallas as pl
from jax.experimental.pallas import tpu as pltpu

N_TOK = 32768
FEAT_DIM = 768
TEXT_DIM = 768
NUM_EXPERTS = 8
HID = 128
BLK = 2048


def _router_body(temp_ref, f_ref, t_ref, fmu_ref, frho_ref, tmu_ref, trho_ref,
                 cmu_ref, crho_ref, ef_ref, et_ref, ec_ref,
                 probs_ref, logits_ref):
    fw = fmu_ref[...] + jnp.log(1.0 + jnp.exp(frho_ref[...])) * ef_ref[...]
    tw = tmu_ref[...] + jnp.log(1.0 + jnp.exp(trho_ref[...])) * et_ref[...]
    cw = cmu_ref[...] + jnp.log(1.0 + jnp.exp(crho_ref[...])) * ec_ref[...]

    fp = jnp.dot(f_ref[...], fw, preferred_element_type=jnp.float32)
    tp = jnp.dot(t_ref[...], tw, preferred_element_type=jnp.float32)
    logits = (jnp.dot(fp, cw[:HID, :], preferred_element_type=jnp.float32)
              + jnp.dot(tp, cw[HID:, :], preferred_element_type=jnp.float32))
    inv_t = 1.0 / jnp.maximum(temp_ref[0, 0], 0.1)
    logits = logits * inv_t
    logits_ref[...] = logits
    m = jnp.max(logits, axis=1, keepdims=True)
    e = jnp.exp(logits - m)
    probs_ref[...] = e / jnp.sum(e, axis=1, keepdims=True)


def kernel(feature, text_embedding, feature_mu, feature_rho, text_mu, text_rho,
           combined_mu, combined_rho, temperature, epsilon_f, epsilon_t, epsilon_c):
    temp2d = temperature.reshape(1, 1)
    full = lambda shape: pl.BlockSpec(shape, lambda i: (0, 0))
    grid = N_TOK // BLK
    probs, logits = pl.pallas_call(
        _router_body,
        grid=(grid,),
        in_specs=[
            full((1, 1)),
            pl.BlockSpec((BLK, FEAT_DIM), lambda i: (i, 0)),
            pl.BlockSpec((BLK, TEXT_DIM), lambda i: (i, 0)),
            full((FEAT_DIM, HID)),
            full((FEAT_DIM, HID)),
            full((TEXT_DIM, HID)),
            full((TEXT_DIM, HID)),
            full((2 * HID, NUM_EXPERTS)),
            full((2 * HID, NUM_EXPERTS)),
            full((FEAT_DIM, HID)),
            full((TEXT_DIM, HID)),
            full((2 * HID, NUM_EXPERTS)),
        ],
        out_specs=[
            pl.BlockSpec((BLK, NUM_EXPERTS), lambda i: (i, 0)),
            pl.BlockSpec((BLK, NUM_EXPERTS), lambda i: (i, 0)),
        ],
        out_shape=[
            jax.ShapeDtypeStruct((N_TOK, NUM_EXPERTS), jnp.float32),
            jax.ShapeDtypeStruct((N_TOK, NUM_EXPERTS), jnp.float32),
        ],
        compiler_params=pltpu.CompilerParams(
            dimension_semantics=("parallel",),
        ),
    )(temp2d, feature, text_embedding, feature_mu, feature_rho, text_mu,
      text_rho, combined_mu, combined_rho, epsilon_f, epsilon_t, epsilon_c)
    return (probs, logits)
